# baseline (speedup 1.0000x reference)
import functools

import jax
import jax.numpy as jnp
from jax import lax
from jax.experimental import pallas as pl
from jax.experimental.pallas import tpu as pltpu
from jax.experimental.pallas import tpu_sc as plsc

N = 32768
DIM = 64
N_SYN = 512
N_SEM = 1024
CTX = 3.0
COMMIT = 0.25
LN_EPS = 1e-5

BLK = 1024


def _ln_rows(x):
    m = jnp.mean(x, axis=-1, keepdims=True)
    v = jnp.mean((x - m) ** 2, axis=-1, keepdims=True)
    return (x - m) / jnp.sqrt(v + LN_EPS)


def _branch(z, cb, W1, b1, g1, be1, W2, b2, Wp, bp, k):
    h = jnp.dot(z, W1, preferred_element_type=jnp.float32) + b1
    h = _ln_rows(h) * g1 + be1
    h = jnp.maximum(h, 0.0)
    h = jnp.maximum(jnp.dot(h, W2, preferred_element_type=jnp.float32) + b2, 0.0)
    ctx = _ln_rows(jnp.dot(h, Wp, preferred_element_type=jnp.float32) + bp)

    zsq = jnp.sum(z * z, axis=1, keepdims=True)
    csq = jnp.sum(cb * cb, axis=1)[None, :]
    d = zsq + csq - 2.0 * jnp.dot(z, cb.T, preferred_element_type=jnp.float32)
    logits = _ln_rows(-jnp.clip(d, 0.0, 10000.0))

    total = logits + CTX * ctx
    idx = jnp.argmax(total, axis=1)
    idx_pure = jnp.argmax(logits, axis=1)

    sel = jax.lax.broadcasted_iota(jnp.int32, (z.shape[0], k), 1) == idx[:, None]
    dsel = jnp.sum(jnp.where(sel, d, 0.0), axis=1)
    loss = ((1.0 + COMMIT) / DIM) * dsel
    flags = (idx != idx_pure).astype(jnp.float32)
    divpart = jnp.sum(flags.reshape(-1, 128), axis=0)[None, :]
    return loss, idx.astype(jnp.int32), divpart


def _vq_kernel(zf_ref, zs_ref, cbsyn_ref, cbsem_ref,
               syn_W1, syn_b1, syn_g1, syn_be1, syn_W2, syn_b2, syn_Wp, syn_bp,
               sem_W1, sem_b1, sem_g1, sem_be1, sem_W2, sem_b2, sem_Wp, sem_bp,
               loss_syn_ref, loss_sem_ref,
               idx_syn_ref, idx_sem_ref, div_syn_ref, div_sem_ref):
    loss_s, idx_s, cnt_s = _branch(
        zf_ref[...], cbsyn_ref[...],
        syn_W1[...], syn_b1[...], syn_g1[...], syn_be1[...],
        syn_W2[...], syn_b2[...], syn_Wp[...], syn_bp[...], N_SYN)
    loss_m, idx_m, cnt_m = _branch(
        zs_ref[...], cbsem_ref[...],
        sem_W1[...], sem_b1[...], sem_g1[...], sem_be1[...],
        sem_W2[...], sem_b2[...], sem_Wp[...], sem_bp[...], N_SEM)

    loss_syn_ref[...] = loss_s
    loss_sem_ref[...] = loss_m
    idx_syn_ref[...] = idx_s
    idx_sem_ref[...] = idx_m

    @pl.when(pl.program_id(0) == 0)
    def _():
        div_syn_ref[...] = jnp.zeros_like(div_syn_ref)
        div_sem_ref[...] = jnp.zeros_like(div_sem_ref)

    div_syn_ref[...] += cnt_s * (1.0 / N)
    div_sem_ref[...] += cnt_m * (1.0 / N)


def _sc_gather(cb_syn, cb_sem, idx_syn, idx_sem):
    info = plsc.get_sparse_core_info()
    nw = info.num_cores * info.num_subcores
    b_per_w = N // nw
    n_chunks = 2
    ch = b_per_w // n_chunks
    mesh = plsc.VectorSubcoreMesh(core_axis_name="c", subcore_axis_name="s")

    cbs_pad = jnp.pad(cb_syn, ((0, 0), (0, 128 - DIM)))
    cbm_pad = jnp.pad(cb_sem, ((0, 0), (0, 128 - DIM)))

    @functools.partial(
        pl.kernel,
        mesh=mesh,
        out_type=[
            jax.ShapeDtypeStruct((N, 128), jnp.float32),
            jax.ShapeDtypeStruct((N, 128), jnp.float32),
        ],
        scratch_types=[
            pltpu.VMEM((ch,), jnp.int32),
            pltpu.VMEM((ch, 128), jnp.float32),
            pltpu.SemaphoreType.DMA,
        ],
    )
    def gather_k(cbs_hbm, cbm_hbm, idxs_hbm, idxm_hbm, zqs_hbm, zqm_hbm,
                 idx_v, rows_v, sem):
        wid = lax.axis_index("s") * info.num_cores + lax.axis_index("c")
        base = wid * b_per_w
        for c in range(n_chunks):
            b = base + c * ch
            pltpu.sync_copy(idxs_hbm.at[pl.ds(b, ch)], idx_v)
            pltpu.async_copy(cbs_hbm.at[idx_v], rows_v, sem).wait()
            pltpu.sync_copy(rows_v, zqs_hbm.at[pl.ds(b, ch)])
            pltpu.sync_copy(idxm_hbm.at[pl.ds(b, ch)], idx_v)
            pltpu.async_copy(cbm_hbm.at[idx_v], rows_v, sem).wait()
            pltpu.sync_copy(rows_v, zqm_hbm.at[pl.ds(b, ch)])

    zqs_pad, zqm_pad = gather_k(cbs_pad, cbm_pad, idx_syn, idx_sem)
    return zqs_pad[:, :DIM], zqm_pad[:, :DIM]


def kernel(z_fast, z_slow, cb_syn, cb_sem,
           syn_W1, syn_b1, syn_g1, syn_be1, syn_W2, syn_b2, syn_Wp, syn_bp,
           sem_W1, sem_b1, sem_g1, sem_be1, sem_W2, sem_b2, sem_Wp, sem_bp,
           update_graph):
    grid = (N // BLK,)
    row_spec = pl.BlockSpec((BLK, DIM), lambda i: (i, 0))
    full = lambda shape: pl.BlockSpec(shape, lambda i: (0,) * len(shape))

    in_specs = [
        row_spec, row_spec,
        full((N_SYN, DIM)), full((N_SEM, DIM)),
        full((DIM, DIM)), full((DIM,)), full((DIM,)), full((DIM,)),
        full((DIM, DIM // 2)), full((DIM // 2,)),
        full((DIM // 2, N_SYN)), full((N_SYN,)),
        full((DIM, DIM)), full((DIM,)), full((DIM,)), full((DIM,)),
        full((DIM, DIM // 2)), full((DIM // 2,)),
        full((DIM // 2, N_SEM)), full((N_SEM,)),
    ]
    out_specs = [
        pl.BlockSpec((BLK,), lambda i: (i,)),
        pl.BlockSpec((BLK,), lambda i: (i,)),
        pl.BlockSpec((BLK,), lambda i: (i,)),
        pl.BlockSpec((BLK,), lambda i: (i,)),
        pl.BlockSpec((1, 128), lambda i: (0, 0)),
        pl.BlockSpec((1, 128), lambda i: (0, 0)),
    ]
    out_shapes = [
        jax.ShapeDtypeStruct((N,), jnp.float32),
        jax.ShapeDtypeStruct((N,), jnp.float32),
        jax.ShapeDtypeStruct((N,), jnp.int32),
        jax.ShapeDtypeStruct((N,), jnp.int32),
        jax.ShapeDtypeStruct((1, 128), jnp.float32),
        jax.ShapeDtypeStruct((1, 128), jnp.float32),
    ]

    (loss_syn, loss_sem, idx_syn, idx_sem,
     div_syn, div_sem) = pl.pallas_call(
        _vq_kernel,
        grid=grid,
        in_specs=in_specs,
        out_specs=out_specs,
        out_shape=out_shapes,
    )(z_fast, z_slow, cb_syn, cb_sem,
      syn_W1, syn_b1, syn_g1, syn_be1, syn_W2, syn_b2, syn_Wp, syn_bp,
      sem_W1, sem_b1, sem_g1, sem_be1, sem_W2, sem_b2, sem_Wp, sem_bp)

    zq_syn, zq_sem = _sc_gather(cb_syn, cb_sem, idx_syn, idx_sem)

    return (zq_syn, zq_sem, loss_syn, loss_sem, idx_syn, idx_sem,
            jnp.sum(div_syn), jnp.sum(div_sem))


# speedup vs baseline: 1.0329x; 1.0329x over previous
import functools

import jax
import jax.numpy as jnp
from jax import lax
from jax.experimental import pallas as pl
from jax.experimental.pallas import tpu as pltpu
from jax.experimental.pallas import tpu_sc as plsc

N = 32768
DIM = 64
N_SYN = 512
N_SEM = 1024
CTX = 3.0
COMMIT = 0.25
LN_EPS = 1e-5

BLK = 1024


def _ln_rows(x):
    m = jnp.mean(x, axis=-1, keepdims=True)
    v = jnp.mean((x - m) ** 2, axis=-1, keepdims=True)
    return (x - m) / jnp.sqrt(v + LN_EPS)


def _branch(z, cb, W1, b1, g1, be1, W2, b2, Wp, bp, k):
    h = jnp.dot(z, W1, preferred_element_type=jnp.float32) + b1
    h = _ln_rows(h) * g1 + be1
    h = jnp.maximum(h, 0.0)
    h = jnp.maximum(jnp.dot(h, W2, preferred_element_type=jnp.float32) + b2, 0.0)
    ctx = _ln_rows(jnp.dot(h, Wp, preferred_element_type=jnp.float32) + bp)

    zsq = jnp.sum(z * z, axis=1, keepdims=True)
    csq = jnp.sum(cb * cb, axis=1)[None, :]
    d = zsq + csq - 2.0 * jnp.dot(z, cb.T, preferred_element_type=jnp.float32)
    logits = _ln_rows(-jnp.clip(d, 0.0, 10000.0))

    total = logits + CTX * ctx
    idx = jnp.argmax(total, axis=1)
    idx_pure = jnp.argmax(logits, axis=1)

    sel = jax.lax.broadcasted_iota(jnp.int32, (z.shape[0], k), 1) == idx[:, None]
    dsel = jnp.sum(jnp.where(sel, d, 0.0), axis=1)
    loss = ((1.0 + COMMIT) / DIM) * dsel
    flags = (idx != idx_pure).astype(jnp.float32)
    divpart = jnp.sum(flags.reshape(-1, 128), axis=0)[None, :]
    return loss, idx.astype(jnp.int32), divpart


def _vq_kernel(zf_ref, zs_ref, cbsyn_ref, cbsem_ref,
               syn_W1, syn_b1, syn_g1, syn_be1, syn_W2, syn_b2, syn_Wp, syn_bp,
               sem_W1, sem_b1, sem_g1, sem_be1, sem_W2, sem_b2, sem_Wp, sem_bp,
               loss_syn_ref, loss_sem_ref,
               idx_syn_ref, idx_sem_ref, div_syn_ref, div_sem_ref):
    loss_s, idx_s, cnt_s = _branch(
        zf_ref[...], cbsyn_ref[...],
        syn_W1[...], syn_b1[...], syn_g1[...], syn_be1[...],
        syn_W2[...], syn_b2[...], syn_Wp[...], syn_bp[...], N_SYN)
    loss_m, idx_m, cnt_m = _branch(
        zs_ref[...], cbsem_ref[...],
        sem_W1[...], sem_b1[...], sem_g1[...], sem_be1[...],
        sem_W2[...], sem_b2[...], sem_Wp[...], sem_bp[...], N_SEM)

    loss_syn_ref[...] = loss_s
    loss_sem_ref[...] = loss_m
    idx_syn_ref[...] = idx_s
    idx_sem_ref[...] = idx_m

    @pl.when(pl.program_id(0) == 0)
    def _():
        div_syn_ref[...] = jnp.zeros_like(div_syn_ref)
        div_sem_ref[...] = jnp.zeros_like(div_sem_ref)

    div_syn_ref[...] += cnt_s * (1.0 / N)
    div_sem_ref[...] += cnt_m * (1.0 / N)


def _sc_gather(cb_syn, cb_sem, idx_syn, idx_sem):
    info = plsc.get_sparse_core_info()
    nw = info.num_cores * info.num_subcores
    b_per_w = N // nw
    n_chunks = 4
    ch = b_per_w // n_chunks
    mesh = plsc.VectorSubcoreMesh(core_axis_name="c", subcore_axis_name="s")

    cbs_pad = jnp.pad(cb_syn, ((0, 0), (0, 128 - DIM)))
    cbm_pad = jnp.pad(cb_sem, ((0, 0), (0, 128 - DIM)))

    @functools.partial(
        pl.kernel,
        mesh=mesh,
        out_type=[
            jax.ShapeDtypeStruct((N, 128), jnp.float32),
            jax.ShapeDtypeStruct((N, 128), jnp.float32),
        ],
        scratch_types=[
            pltpu.VMEM((ch,), jnp.int32),
            pltpu.VMEM((ch,), jnp.int32),
            pltpu.VMEM((ch, 128), jnp.float32),
            pltpu.VMEM((ch, 128), jnp.float32),
            pltpu.SemaphoreType.DMA,
            pltpu.SemaphoreType.DMA,
        ],
    )
    def gather_k(cbs_hbm, cbm_hbm, idxs_hbm, idxm_hbm, zqs_hbm, zqm_hbm,
                 idx_v0, idx_v1, rows_v0, rows_v1, sem0, sem1):
        wid = lax.axis_index("s") * info.num_cores + lax.axis_index("c")
        base = wid * b_per_w
        tasks = []
        for c in range(n_chunks):
            tasks.append((idxs_hbm, cbs_hbm, zqs_hbm, c * ch))
            tasks.append((idxm_hbm, cbm_hbm, zqm_hbm, c * ch))
        idx_bufs = (idx_v0, idx_v1)
        row_bufs = (rows_v0, rows_v1)
        sems = (sem0, sem1)
        copies = [None, None]
        for t, (idx_hbm, cb_hbm, zq_hbm, off) in enumerate(tasks):
            s = t % 2
            pltpu.sync_copy(idx_hbm.at[pl.ds(base + off, ch)], idx_bufs[s])
            copies[s] = pltpu.async_copy(cb_hbm.at[idx_bufs[s]], row_bufs[s],
                                         sems[s])
            if t > 0:
                p, (_, _, pzq, poff) = (t - 1) % 2, tasks[t - 1]
                copies[p].wait()
                pltpu.sync_copy(row_bufs[p], pzq.at[pl.ds(base + poff, ch)])
        t = len(tasks) - 1
        copies[t % 2].wait()
        pltpu.sync_copy(row_bufs[t % 2],
                        tasks[t][2].at[pl.ds(base + tasks[t][3], ch)])

    zqs_pad, zqm_pad = gather_k(cbs_pad, cbm_pad, idx_syn, idx_sem)
    return zqs_pad[:, :DIM], zqm_pad[:, :DIM]


def kernel(z_fast, z_slow, cb_syn, cb_sem,
           syn_W1, syn_b1, syn_g1, syn_be1, syn_W2, syn_b2, syn_Wp, syn_bp,
           sem_W1, sem_b1, sem_g1, sem_be1, sem_W2, sem_b2, sem_Wp, sem_bp,
           update_graph):
    grid = (N // BLK,)
    row_spec = pl.BlockSpec((BLK, DIM), lambda i: (i, 0))
    full = lambda shape: pl.BlockSpec(shape, lambda i: (0,) * len(shape))

    in_specs = [
        row_spec, row_spec,
        full((N_SYN, DIM)), full((N_SEM, DIM)),
        full((DIM, DIM)), full((DIM,)), full((DIM,)), full((DIM,)),
        full((DIM, DIM // 2)), full((DIM // 2,)),
        full((DIM // 2, N_SYN)), full((N_SYN,)),
        full((DIM, DIM)), full((DIM,)), full((DIM,)), full((DIM,)),
        full((DIM, DIM // 2)), full((DIM // 2,)),
        full((DIM // 2, N_SEM)), full((N_SEM,)),
    ]
    out_specs = [
        pl.BlockSpec((BLK,), lambda i: (i,)),
        pl.BlockSpec((BLK,), lambda i: (i,)),
        pl.BlockSpec((BLK,), lambda i: (i,)),
        pl.BlockSpec((BLK,), lambda i: (i,)),
        pl.BlockSpec((1, 128), lambda i: (0, 0)),
        pl.BlockSpec((1, 128), lambda i: (0, 0)),
    ]
    out_shapes = [
        jax.ShapeDtypeStruct((N,), jnp.float32),
        jax.ShapeDtypeStruct((N,), jnp.float32),
        jax.ShapeDtypeStruct((N,), jnp.int32),
        jax.ShapeDtypeStruct((N,), jnp.int32),
        jax.ShapeDtypeStruct((1, 128), jnp.float32),
        jax.ShapeDtypeStruct((1, 128), jnp.float32),
    ]

    (loss_syn, loss_sem, idx_syn, idx_sem,
     div_syn, div_sem) = pl.pallas_call(
        _vq_kernel,
        grid=grid,
        in_specs=in_specs,
        out_specs=out_specs,
        out_shape=out_shapes,
    )(z_fast, z_slow, cb_syn, cb_sem,
      syn_W1, syn_b1, syn_g1, syn_be1, syn_W2, syn_b2, syn_Wp, syn_bp,
      sem_W1, sem_b1, sem_g1, sem_be1, sem_W2, sem_b2, sem_Wp, sem_bp)

    zq_syn, zq_sem = _sc_gather(cb_syn, cb_sem, idx_syn, idx_sem)

    return (zq_syn, zq_sem, loss_syn, loss_sem, idx_syn, idx_sem,
            jnp.sum(div_syn), jnp.sum(div_sem))
